# baseline (speedup 1.0000x reference)
import functools

import jax
import jax.numpy as jnp
from jax import lax
from jax.experimental import pallas as pl
from jax.experimental.pallas import tpu as pltpu
from jax.experimental.pallas import tpu_sc as plsc

_BT = 128
_HC = 8
_PL = 129


@functools.partial(jax.jit, static_argnums=(2, 3))
def _gather_sc(idx_p, table, h, n_workers):
    b, hp = idx_p.shape
    _, d = table.shape
    n_bt = b // _BT
    bt_per_w = n_bt // n_workers
    n_hchunk = hp // _HC
    db = d // 8

    mesh = plsc.VectorSubcoreMesh(core_axis_name="c", subcore_axis_name="s")

    @functools.partial(
        pl.kernel,
        mesh=mesh,
        compiler_params=pltpu.CompilerParams(
            use_tc_tiling_on_sc=False, needs_layout_passes=False
        ),
        out_type=jax.ShapeDtypeStruct((h, db, n_bt, 8, _BT), jnp.float32),
        scratch_types=[
            pltpu.VMEM((_BT, hp + 1), jnp.int32),
            pltpu.VMEM((_HC, _BT), jnp.int32),
            pltpu.VMEM((_HC, _BT, d), jnp.float32),
            pltpu.VMEM((_HC, db, 8, _PL), jnp.float32),
            pltpu.SemaphoreType.DMA,
        ],
    )
    def k(idx_hbm, table_hbm, out_hbm, idx_v, idx_t, gbuf, tbuf, gsem):
        nc = 2
        wid = lax.axis_index("s") * nc + lax.axis_index("c")
        bt0 = wid * bt_per_w
        lanes = lax.iota(jnp.int32, 16)
        band_lo = lax.shift_right_logical(lanes, 3)
        band_hi = band_lo + 2
        ds_v = lanes & 7

        def bt_body(t, carry):
            bt = bt0 + t
            pltpu.sync_copy(
                idx_hbm.at[pl.ds(bt * _BT, _BT), :],
                idx_v.at[:, pl.ds(0, hp)],
            )

            for hc in range(n_hchunk):
                h0 = hc * _HC
                hn = min(_HC, h - h0)
                if hn <= 0:
                    break

                def idx_tr(hh, carry2, h0=h0):
                    hv = jnp.full((16,), h0, jnp.int32) + hh
                    for g in range(_BT // 16):
                        v = plsc.load_gather(idx_v, [g * 16 + lanes, hv])
                        idx_t[hh, pl.ds(g * 16, 16)] = v
                    return carry2

                lax.fori_loop(0, _HC, idx_tr, 0)

                copies = []
                for hh in range(_HC):
                    copies.append(
                        pltpu.async_copy(
                            table_hbm.at[idx_t.at[hh]], gbuf.at[hh], gsem
                        )
                    )
                for cp in copies:
                    cp.wait()

                @plsc.parallel_loop(0, _HC * d, unroll=2)
                def tr_body(i):
                    hh = i // d
                    r = i % d
                    hv = jnp.full((16,), 0, jnp.int32) + hh
                    rv = jnp.full((16,), 0, jnp.int32) + r
                    for g in range(_BT // 16):
                        v = plsc.load_gather(gbuf, [hv, g * 16 + lanes, rv])
                        tbuf[hh, r // 8, r % 8, pl.ds(g * 16, 16)] = v

                pltpu.sync_copy(
                    tbuf.at[pl.ds(0, hn), :, :, pl.ds(0, _BT)],
                    out_hbm.at[pl.ds(h0, hn), :, bt],
                )
            return carry

        lax.fori_loop(0, bt_per_w, bt_body, 0)

    return k(idx_p, table)


def kernel(indices, table):
    b, h = indices.shape
    _, d = table.shape
    info = plsc.get_sparse_core_info()
    n_workers = info.num_cores * info.num_subcores
    hp = (h + 7) // 8 * 8
    idx_p = jnp.pad(indices, ((0, 0), (0, hp - h)))
    out5d = _gather_sc(idx_p, table, h, n_workers)
    return out5d.transpose(2, 4, 0, 1, 3).reshape(b, h, d)


# speedup vs baseline: 1.7338x; 1.7338x over previous
import functools

import jax
import jax.numpy as jnp
from jax import lax
from jax.experimental import pallas as pl
from jax.experimental.pallas import tpu as pltpu
from jax.experimental.pallas import tpu_sc as plsc

_CHUNK_B = 16


@functools.partial(jax.jit, static_argnums=(2,))
def _gather_sc(indices, table, n_workers):
    b, h = indices.shape
    _, d = table.shape
    b_per_w = b // n_workers
    chunks_per_worker = b_per_w // _CHUNK_B

    mesh = plsc.VectorSubcoreMesh(core_axis_name="c", subcore_axis_name="s")

    @functools.partial(
        pl.kernel,
        mesh=mesh,
        compiler_params=pltpu.CompilerParams(use_tc_tiling_on_sc=False),
        out_type=jax.ShapeDtypeStruct((b, h, d), jnp.float32),
        scratch_types=[
            pltpu.VMEM((_CHUNK_B, h), jnp.int32),
            pltpu.VMEM((_CHUNK_B, h, d), jnp.float32),
            pltpu.SemaphoreType.DMA,
        ],
    )
    def k(idx_hbm, table_hbm, out_hbm, idx_v, rows_v, sem):
        nc = 2
        wid = lax.axis_index("s") * nc + lax.axis_index("c")
        b0 = wid * b_per_w

        def chunk_body(i, carry):
            base = b0 + i * _CHUNK_B
            pltpu.sync_copy(idx_hbm.at[pl.ds(base, _CHUNK_B)], idx_v)
            copies = []
            for j in range(_CHUNK_B):
                copies.append(
                    pltpu.async_copy(
                        table_hbm.at[idx_v.at[j]],
                        rows_v.at[j],
                        sem,
                    )
                )
            for c in copies:
                c.wait()
            pltpu.sync_copy(rows_v, out_hbm.at[pl.ds(base, _CHUNK_B)])
            return carry

        lax.fori_loop(0, chunks_per_worker, chunk_body, 0)

    return k(indices, table)


def kernel(indices, table):
    info = plsc.get_sparse_core_info()
    n_workers = info.num_cores * info.num_subcores
    return _gather_sc(indices, table, n_workers)
